# Initial kernel scaffold; baseline (speedup 1.0000x reference)
import jax
import jax.numpy as jnp
from jax.experimental import pallas as pl


def kernel(latent, condition, edge_index, W1, b1, W2, b2, W3, b3, W4, b4):
    raise NotImplementedError("write your pallas kernel here")


# speedup vs baseline: 21.2139x; 21.2139x over previous
import jax
import jax.numpy as jnp
from jax import lax
from jax.experimental import pallas as pl
from jax.experimental.pallas import tpu as pltpu
from jax.experimental.pallas import tpu_sc as plsc

N = 10000
E = 320000
D = 128
NP = 10240
NC = 2
NS = 16
NW = NC * NS
CHUNK = 128
NCHUNKS = 2560
EP = NCHUNKS * CHUNK
CPT = NCHUNKS // NW
G = 8
NG = CPT // G
NBUF = 4
ROWS_PER_TILE = NP // NS
TILE_WB = ROWS_PER_TILE // CHUNK


def _sc_mesh():
    return plsc.VectorSubcoreMesh(
        core_axis_name="c", subcore_axis_name="s", num_cores=NC, num_subcores=NS
    )


def _deg_sc(dst_hbm, out_hbm, dst_v, ones_v, zrow_v, acc_sh):
    c = lax.axis_index("c")
    s = lax.axis_index("s")
    wid = s * NC + c
    pltpu.sync_copy(dst_hbm.at[pl.ds(wid * CPT, CPT)], dst_v)

    z = jnp.zeros((16,), jnp.float32)
    one = jnp.ones((16,), jnp.float32)
    for k in range(8):
        ones_v[pl.ds(k * 16, 16)] = one

    def zrow(i, _):
        zrow_v[pl.ds(i * 16, 16)] = z
        return 0

    lax.fori_loop(0, ROWS_PER_TILE // 16, zrow, 0)
    pltpu.sync_copy(zrow_v, acc_sh.at[pl.ds(s * ROWS_PER_TILE, ROWS_PER_TILE)])
    plsc.subcore_barrier()

    def chunk(j, _):
        pltpu.sync_copy(ones_v, acc_sh.at[dst_v.at[j]], add=True)
        return 0

    lax.fori_loop(0, CPT, chunk, 0)
    plsc.subcore_barrier()
    pltpu.sync_copy(
        acc_sh.at[pl.ds(s * ROWS_PER_TILE, ROWS_PER_TILE)],
        out_hbm.at[c, pl.ds(s * ROWS_PER_TILE, ROWS_PER_TILE)],
    )


def _degree(dst2d):
    return pl.kernel(
        _deg_sc,
        out_type=jax.ShapeDtypeStruct((NC, NP), jnp.float32),
        mesh=_sc_mesh(),
        scratch_types=[
            pltpu.VMEM((CPT, CHUNK), jnp.int32),
            pltpu.VMEM((CHUNK,), jnp.float32),
            pltpu.VMEM((ROWS_PER_TILE,), jnp.float32),
            pltpu.VMEM_SHARED((NP,), jnp.float32),
        ],
    )(dst2d)


def _prop_sc(y_hbm, src_hbm, dst_hbm, out_hbm, srcb, dstb, bufs, sems, isems,
             acc_sh):
    c = lax.axis_index("c")
    s = lax.axis_index("s")
    wid = s * NC + c
    base = wid * CPT

    z = jnp.zeros((16,), jnp.float32)

    def zrow(i, _):
        for k in range(8):
            bufs[0][i, pl.ds(k * 16, 16)] = z
        return 0

    lax.fori_loop(0, CHUNK, zrow, 0)
    for t in range(TILE_WB):
        pltpu.sync_copy(
            bufs[0], acc_sh.at[pl.ds(s * ROWS_PER_TILE + t * CHUNK, CHUNK)]
        )
    plsc.subcore_barrier()

    pltpu.sync_copy(src_hbm.at[pl.ds(base, G)], srcb[0])
    pltpu.sync_copy(dst_hbm.at[pl.ds(base, G)], dstb[0])
    pltpu.async_copy(src_hbm.at[pl.ds(base + G, G)], srcb[1], isems[1])
    pltpu.async_copy(dst_hbm.at[pl.ds(base + G, G)], dstb[1], isems[1])
    for b in range(2):
        pltpu.async_copy(y_hbm.at[srcb[0].at[b]], bufs[b], sems[b])

    def pair(g2, _):
        for par in range(2):
            g = g2 * 2 + par
            for i in range(G):
                b = i % 2
                pltpu.make_async_copy(
                    y_hbm.at[srcb[par].at[i]], bufs[b], sems[b]
                ).wait()
                if i == G - 2:
                    def iwait():
                        pltpu.make_async_copy(
                            src_hbm.at[pl.ds(0, G)], srcb[1 - par],
                            isems[1 - par],
                        ).wait()
                        pltpu.make_async_copy(
                            dst_hbm.at[pl.ds(0, G)], dstb[1 - par],
                            isems[1 - par],
                        ).wait()
                    if par == 0:
                        iwait()
                    else:
                        pl.when(g2 < NG // 2 - 1)(iwait)
                pltpu.sync_copy(bufs[b], acc_sh.at[dstb[par].at[i]], add=True)
                if i < G - 2:
                    pltpu.async_copy(y_hbm.at[srcb[par].at[i + 2]], bufs[b],
                                     sems[b])
                else:
                    def nxt(b=b, i=i, par=par):
                        pltpu.async_copy(
                            y_hbm.at[srcb[1 - par].at[i + 2 - G]], bufs[b],
                            sems[b],
                        )
                    if par == 0:
                        nxt()
                    else:
                        pl.when(g2 < NG // 2 - 1)(nxt)
            def ild(par=par, g=g):
                pltpu.async_copy(
                    src_hbm.at[pl.ds(base + (g + 2) * G, G)], srcb[par],
                    isems[par],
                )
                pltpu.async_copy(
                    dst_hbm.at[pl.ds(base + (g + 2) * G, G)], dstb[par],
                    isems[par],
                )
            pl.when(g2 < NG // 2 - 1)(ild)
        return 0

    lax.fori_loop(0, NG // 2, pair, 0)
    plsc.subcore_barrier()
    for t in range(TILE_WB):
        r0 = s * ROWS_PER_TILE + t * CHUNK
        pltpu.sync_copy(
            acc_sh.at[pl.ds(r0, CHUNK)], out_hbm.at[c, pl.ds(r0, CHUNK)]
        )


def _prop(y, src2d, dst2d):
    return pl.kernel(
        _prop_sc,
        out_type=jax.ShapeDtypeStruct((NC, NP, D), jnp.float32),
        mesh=_sc_mesh(),
        scratch_types=[
            [pltpu.VMEM((G, CHUNK), jnp.int32) for _ in range(2)],
            [pltpu.VMEM((G, CHUNK), jnp.int32) for _ in range(2)],
            [pltpu.VMEM((CHUNK, D), jnp.float32) for _ in range(2)],
            [pltpu.SemaphoreType.DMA for _ in range(2)],
            [pltpu.SemaphoreType.DMA for _ in range(2)],
            pltpu.VMEM_SHARED((NP, D), jnp.float32),
        ],
    )(y, src2d, dst2d)


def _sprop_sc(val_hbm, src_hbm, dst_hbm, out_hbm, src_v, dst_v, bufs, sems,
              zrow_v, acc_sh):
    c = lax.axis_index("c")
    s = lax.axis_index("s")
    wid = s * NC + c
    pltpu.sync_copy(src_hbm.at[pl.ds(wid * CPT, CPT)], src_v)
    pltpu.sync_copy(dst_hbm.at[pl.ds(wid * CPT, CPT)], dst_v)

    z = jnp.zeros((16,), jnp.float32)

    def zrow(i, _):
        zrow_v[pl.ds(i * 16, 16)] = z
        return 0

    lax.fori_loop(0, ROWS_PER_TILE // 16, zrow, 0)
    pltpu.sync_copy(zrow_v, acc_sh.at[pl.ds(s * ROWS_PER_TILE, ROWS_PER_TILE)])
    plsc.subcore_barrier()

    for b in range(NBUF):
        pltpu.async_copy(val_hbm.at[src_v.at[b]], bufs[b], sems[b])

    def group(g, _):
        for b in range(NBUF):
            j = g * NBUF + b
            pltpu.make_async_copy(val_hbm.at[src_v.at[j]], bufs[b], sems[b]).wait()
            pltpu.sync_copy(bufs[b], acc_sh.at[dst_v.at[j]], add=True)
            pltpu.async_copy(val_hbm.at[src_v.at[j + NBUF]], bufs[b], sems[b])
        return 0

    lax.fori_loop(0, (CPT - NBUF) // NBUF, group, 0)
    for b in range(NBUF):
        j = CPT - NBUF + b
        pltpu.make_async_copy(val_hbm.at[src_v.at[j]], bufs[b], sems[b]).wait()
        pltpu.sync_copy(bufs[b], acc_sh.at[dst_v.at[j]], add=True)

    plsc.subcore_barrier()
    pltpu.sync_copy(
        acc_sh.at[pl.ds(s * ROWS_PER_TILE, ROWS_PER_TILE)],
        out_hbm.at[c, pl.ds(s * ROWS_PER_TILE, ROWS_PER_TILE)],
    )


def _sprop(val, src2d, dst2d):
    return pl.kernel(
        _sprop_sc,
        out_type=jax.ShapeDtypeStruct((NC, NP), jnp.float32),
        mesh=_sc_mesh(),
        scratch_types=[
            pltpu.VMEM((CPT, CHUNK), jnp.int32),
            pltpu.VMEM((CPT, CHUNK), jnp.int32),
            [pltpu.VMEM((CHUNK,), jnp.float32) for _ in range(NBUF)],
            [pltpu.SemaphoreType.DMA for _ in range(NBUF)],
            pltpu.VMEM((ROWS_PER_TILE,), jnp.float32),
            pltpu.VMEM_SHARED((NP,), jnp.float32),
        ],
    )(val, src2d, dst2d)


BLK = 1024
GRID = NP // BLK


def _row_spec():
    return pl.BlockSpec((BLK, D), lambda i: (i, 0))


def _part_spec():
    return pl.BlockSpec((NC, BLK, D), lambda i: (0, i, 0))


def _vec_spec():
    return pl.BlockSpec((BLK,), lambda i: (i,))


def _w_spec():
    return pl.BlockSpec((D, D), lambda i: (0, 0))


def _bias_spec():
    return pl.BlockSpec((1, D), lambda i: (0, 0))


def _pvec_spec():
    return pl.BlockSpec((NC, BLK), lambda i: (0, i))


def _prep_tc(lat, cond, w1, w2, w3a, w3b, degp, m, y1, dinv, q):
    dp = degp[...]
    deg = dp[0] + dp[1] + 1.0
    di = lax.rsqrt(deg)
    dinv[...] = di
    q[...] = 1.0 / deg
    wa = jnp.dot(w1[...], w3a[...], preferred_element_type=jnp.float32)
    wb = jnp.dot(w2[...], w3b[...], preferred_element_type=jnp.float32)
    mv = jnp.dot(lat[...], wa, preferred_element_type=jnp.float32) + jnp.dot(
        cond[...], wb, preferred_element_type=jnp.float32
    )
    m[...] = mv
    y1[...] = mv * di[:, None]


def _prep(lat, cond, w1, w2, w3a, w3b, degp):
    return pl.pallas_call(
        _prep_tc,
        grid=(GRID,),
        in_specs=[
            _row_spec(), _row_spec(), _w_spec(), _w_spec(), _w_spec(),
            _w_spec(), _pvec_spec(),
        ],
        out_specs=[_row_spec(), _row_spec(), _vec_spec(), _vec_spec()],
        out_shape=[
            jax.ShapeDtypeStruct((NP, D), jnp.float32),
            jax.ShapeDtypeStruct((NP, D), jnp.float32),
            jax.ShapeDtypeStruct((NP,), jnp.float32),
            jax.ShapeDtypeStruct((NP,), jnp.float32),
        ],
    )(lat, cond, w1, w2, w3a, w3b, degp)


def _mid1_tc(p, sp, m, dinv, q, a1, y2, s1, ds1):
    di = dinv[...]
    qq = q[...]
    pv = p[...]
    spv = sp[...]
    a = di[:, None] * (pv[0] + pv[1]) + qq[:, None] * m[...]
    a1[...] = a
    y2[...] = a * di[:, None]
    sv = di * (spv[0] + spv[1]) + qq
    s1[...] = sv
    ds1[...] = di * sv


def _mid1(p, sp, m, dinv, q):
    return pl.pallas_call(
        _mid1_tc,
        grid=(GRID,),
        in_specs=[_part_spec(), _pvec_spec(), _row_spec(), _vec_spec(),
                  _vec_spec()],
        out_specs=[_row_spec(), _row_spec(), _vec_spec(), _vec_spec()],
        out_shape=[
            jax.ShapeDtypeStruct((NP, D), jnp.float32),
            jax.ShapeDtypeStruct((NP, D), jnp.float32),
            jax.ShapeDtypeStruct((NP,), jnp.float32),
            jax.ShapeDtypeStruct((NP,), jnp.float32),
        ],
    )(p, sp, m, dinv, q)


def _mid2_tc(p, sp, a1, s1, dinv, q, a2, y3, s2):
    di = dinv[...]
    qq = q[...]
    pv = p[...]
    spv = sp[...]
    a = di[:, None] * (pv[0] + pv[1]) + qq[:, None] * a1[...]
    a2[...] = a
    y3[...] = a * di[:, None]
    s2[...] = di * (spv[0] + spv[1]) + qq * s1[...]


def _mid2(p, sp, a1, s1, dinv, q):
    return pl.pallas_call(
        _mid2_tc,
        grid=(GRID,),
        in_specs=[_part_spec(), _pvec_spec(), _row_spec(), _vec_spec(),
                  _vec_spec(), _vec_spec()],
        out_specs=[_row_spec(), _row_spec(), _vec_spec()],
        out_shape=[
            jax.ShapeDtypeStruct((NP, D), jnp.float32),
            jax.ShapeDtypeStruct((NP, D), jnp.float32),
            jax.ShapeDtypeStruct((NP,), jnp.float32),
        ],
    )(p, sp, a1, s1, dinv, q)


def _final_tc(p, a2, s1, s2, dinv, q, w3a, w3b, w4, b1, b2, b3, b4, out):
    di = dinv[...]
    qq = q[...]
    pv = p[...]
    a3 = di[:, None] * (pv[0] + pv[1]) + qq[:, None] * a2[...]
    w4v = w4[...]
    bc = jnp.dot(b1[...], w3a[...], preferred_element_type=jnp.float32) + jnp.dot(
        b2[...], w3b[...], preferred_element_type=jnp.float32
    )
    bcw4 = jnp.dot(bc, w4v, preferred_element_type=jnp.float32)
    b3w4 = jnp.dot(b3[...], w4v, preferred_element_type=jnp.float32)
    out[...] = (
        jnp.dot(a3, w4v, preferred_element_type=jnp.float32)
        + s2[...][:, None] * bcw4
        + s1[...][:, None] * b3w4
        + b4[...]
    )


def _final(p, a2, s1, s2, dinv, q, w3a, w3b, w4, b1, b2, b3, b4):
    return pl.pallas_call(
        _final_tc,
        grid=(GRID,),
        in_specs=[
            _part_spec(), _row_spec(), _vec_spec(), _vec_spec(), _vec_spec(),
            _vec_spec(), _w_spec(), _w_spec(), _w_spec(), _bias_spec(),
            _bias_spec(), _bias_spec(), _bias_spec(),
        ],
        out_specs=_row_spec(),
        out_shape=jax.ShapeDtypeStruct((NP, D), jnp.float32),
    )(p, a2, s1, s2, dinv, q, w3a, w3b, w4, b1, b2, b3, b4)


def kernel(latent, condition, edge_index, W1, b1, W2, b2, W3, b3, W4, b4):
    src = edge_index[0]
    dst = edge_index[1]
    pad_idx = (N + jnp.arange(EP - E, dtype=jnp.int32) % (NP - N)).astype(jnp.int32)
    src2d = jnp.concatenate([src, pad_idx]).reshape(NCHUNKS, CHUNK)
    dst2d = jnp.concatenate([dst, pad_idx]).reshape(NCHUNKS, CHUNK)

    lat = jnp.zeros((NP, D), jnp.float32).at[:N].set(latent)
    cond = jnp.zeros((NP, D), jnp.float32).at[:N].set(condition)
    b1r = b1.reshape(1, D)
    b2r = b2.reshape(1, D)
    b3r = b3.reshape(1, D)
    b4r = b4.reshape(1, D)
    w3a = W3[:D]
    w3b = W3[D:]

    degp = _degree(dst2d)
    m, y1, dinv, q = _prep(lat, cond, W1, W2, w3a, w3b, degp)
    p1 = _prop(y1, src2d, dst2d)
    sp1 = _sprop(dinv, src2d, dst2d)
    a1, y2, s1, ds1 = _mid1(p1, sp1, m, dinv, q)
    p2 = _prop(y2, src2d, dst2d)
    sp2 = _sprop(ds1, src2d, dst2d)
    a2, y3, s2 = _mid2(p2, sp2, a1, s1, dinv, q)
    p3 = _prop(y3, src2d, dst2d)
    out = _final(p3, a2, s1, s2, dinv, q, w3a, w3b, W4, b1r, b2r, b3r, b4r)
    return out[:N]


# speedup vs baseline: 25.0776x; 1.1821x over previous
import jax
import jax.numpy as jnp
from jax import lax
from jax.experimental import pallas as pl
from jax.experimental.pallas import tpu as pltpu
from jax.experimental.pallas import tpu_sc as plsc

N = 10000
E = 320000
D = 128
NP = 10240
NC = 2
NS = 16
NW = NC * NS
CHUNK = 128
NCHUNKS = 2560
EP = NCHUNKS * CHUNK
CPT = NCHUNKS // NW
G = 8
NG = CPT // G
NBUF = 4
ROWS_PER_TILE = NP // NS
TILE_WB = ROWS_PER_TILE // CHUNK


def _sc_mesh():
    return plsc.VectorSubcoreMesh(
        core_axis_name="c", subcore_axis_name="s", num_cores=NC, num_subcores=NS
    )


def _deg_sc(dst_hbm, out_hbm, dst_v, ones_v, zrow_v, acc_sh):
    c = lax.axis_index("c")
    s = lax.axis_index("s")
    wid = s * NC + c
    pltpu.sync_copy(dst_hbm.at[pl.ds(wid * CPT, CPT)], dst_v)

    z = jnp.zeros((16,), jnp.float32)
    one = jnp.ones((16,), jnp.float32)
    for k in range(8):
        ones_v[pl.ds(k * 16, 16)] = one

    def zrow(i, _):
        zrow_v[pl.ds(i * 16, 16)] = z
        return 0

    lax.fori_loop(0, ROWS_PER_TILE // 16, zrow, 0)
    pltpu.sync_copy(zrow_v, acc_sh.at[pl.ds(s * ROWS_PER_TILE, ROWS_PER_TILE)])
    plsc.subcore_barrier()

    def chunk(j, _):
        pltpu.sync_copy(ones_v, acc_sh.at[dst_v.at[j]], add=True)
        return 0

    lax.fori_loop(0, CPT, chunk, 0)
    plsc.subcore_barrier()
    pltpu.sync_copy(
        acc_sh.at[pl.ds(s * ROWS_PER_TILE, ROWS_PER_TILE)],
        out_hbm.at[c, pl.ds(s * ROWS_PER_TILE, ROWS_PER_TILE)],
    )


def _degree(dst2d):
    return pl.kernel(
        _deg_sc,
        out_type=jax.ShapeDtypeStruct((NC, NP), jnp.float32),
        mesh=_sc_mesh(),
        scratch_types=[
            pltpu.VMEM((CPT, CHUNK), jnp.int32),
            pltpu.VMEM((CHUNK,), jnp.float32),
            pltpu.VMEM((ROWS_PER_TILE,), jnp.float32),
            pltpu.VMEM_SHARED((NP,), jnp.float32),
        ],
    )(dst2d)


def _make_prop_body(with_s):
    def body(y_hbm, *args):
        if with_s:
            (val_hbm, src_hbm, dst_hbm, out_hbm, sout_hbm, srcb, dstb, bufs,
             sems, isems, sbufs, ssems, zvec, acc_sh, sacc_sh) = args
        else:
            (src_hbm, dst_hbm, out_hbm, srcb, dstb, bufs, sems, isems,
             acc_sh) = args
        c = lax.axis_index("c")
        s = lax.axis_index("s")
        wid = s * NC + c
        base = wid * CPT

        z = jnp.zeros((16,), jnp.float32)

        def zrow(i, _):
            for k in range(8):
                bufs[0][i, pl.ds(k * 16, 16)] = z
            return 0

        lax.fori_loop(0, CHUNK, zrow, 0)
        for t in range(TILE_WB):
            pltpu.sync_copy(
                bufs[0], acc_sh.at[pl.ds(s * ROWS_PER_TILE + t * CHUNK, CHUNK)]
            )
        if with_s:
            def zv(i, _):
                zvec[pl.ds(i * 16, 16)] = z
                return 0

            lax.fori_loop(0, ROWS_PER_TILE // 16, zv, 0)
            pltpu.sync_copy(
                zvec, sacc_sh.at[pl.ds(s * ROWS_PER_TILE, ROWS_PER_TILE)]
            )
        plsc.subcore_barrier()

        pltpu.sync_copy(src_hbm.at[pl.ds(base, G)], srcb[0])
        pltpu.sync_copy(dst_hbm.at[pl.ds(base, G)], dstb[0])
        pltpu.async_copy(src_hbm.at[pl.ds(base + G, G)], srcb[1], isems[1])
        pltpu.async_copy(dst_hbm.at[pl.ds(base + G, G)], dstb[1], isems[1])
        for b in range(2):
            pltpu.async_copy(y_hbm.at[srcb[0].at[b]], bufs[b], sems[b])
            if with_s:
                pltpu.async_copy(val_hbm.at[srcb[0].at[b]], sbufs[b], ssems[b])

        def pair(g2, _):
            for par in range(2):
                g = g2 * 2 + par
                for i in range(G):
                    b = i % 2
                    pltpu.make_async_copy(
                        y_hbm.at[srcb[par].at[i]], bufs[b], sems[b]
                    ).wait()
                    if with_s:
                        pltpu.make_async_copy(
                            val_hbm.at[srcb[par].at[i]], sbufs[b], ssems[b]
                        ).wait()
                    if i == G - 2:
                        def iwait():
                            pltpu.make_async_copy(
                                src_hbm.at[pl.ds(0, G)], srcb[1 - par],
                                isems[1 - par],
                            ).wait()
                            pltpu.make_async_copy(
                                dst_hbm.at[pl.ds(0, G)], dstb[1 - par],
                                isems[1 - par],
                            ).wait()
                        if par == 0:
                            iwait()
                        else:
                            pl.when(g2 < NG // 2 - 1)(iwait)
                    pltpu.sync_copy(bufs[b], acc_sh.at[dstb[par].at[i]],
                                    add=True)
                    if with_s:
                        pltpu.sync_copy(sbufs[b], sacc_sh.at[dstb[par].at[i]],
                                        add=True)
                    if i < G - 2:
                        pltpu.async_copy(y_hbm.at[srcb[par].at[i + 2]],
                                         bufs[b], sems[b])
                        if with_s:
                            pltpu.async_copy(val_hbm.at[srcb[par].at[i + 2]],
                                             sbufs[b], ssems[b])
                    else:
                        def nxt(b=b, i=i, par=par):
                            pltpu.async_copy(
                                y_hbm.at[srcb[1 - par].at[i + 2 - G]], bufs[b],
                                sems[b],
                            )
                            if with_s:
                                pltpu.async_copy(
                                    val_hbm.at[srcb[1 - par].at[i + 2 - G]],
                                    sbufs[b], ssems[b],
                                )
                        if par == 0:
                            nxt()
                        else:
                            pl.when(g2 < NG // 2 - 1)(nxt)
                def ild(par=par, g=g):
                    pltpu.async_copy(
                        src_hbm.at[pl.ds(base + (g + 2) * G, G)], srcb[par],
                        isems[par],
                    )
                    pltpu.async_copy(
                        dst_hbm.at[pl.ds(base + (g + 2) * G, G)], dstb[par],
                        isems[par],
                    )
                pl.when(g2 < NG // 2 - 1)(ild)
            return 0

        lax.fori_loop(0, NG // 2, pair, 0)
        plsc.subcore_barrier()
        for t in range(TILE_WB):
            r0 = s * ROWS_PER_TILE + t * CHUNK
            pltpu.sync_copy(
                acc_sh.at[pl.ds(r0, CHUNK)], out_hbm.at[c, pl.ds(r0, CHUNK)]
            )
        if with_s:
            pltpu.sync_copy(
                sacc_sh.at[pl.ds(s * ROWS_PER_TILE, ROWS_PER_TILE)],
                sout_hbm.at[c, pl.ds(s * ROWS_PER_TILE, ROWS_PER_TILE)],
            )

    return body


def _prop(y, src2d, dst2d):
    return pl.kernel(
        _make_prop_body(False),
        out_type=jax.ShapeDtypeStruct((NC, NP, D), jnp.float32),
        mesh=_sc_mesh(),
        scratch_types=[
            [pltpu.VMEM((G, CHUNK), jnp.int32) for _ in range(2)],
            [pltpu.VMEM((G, CHUNK), jnp.int32) for _ in range(2)],
            [pltpu.VMEM((CHUNK, D), jnp.float32) for _ in range(2)],
            [pltpu.SemaphoreType.DMA for _ in range(2)],
            [pltpu.SemaphoreType.DMA for _ in range(2)],
            pltpu.VMEM_SHARED((NP, D), jnp.float32),
        ],
    )(y, src2d, dst2d)


def _prop_s(y, val, src2d, dst2d):
    return pl.kernel(
        _make_prop_body(True),
        out_type=[
            jax.ShapeDtypeStruct((NC, NP, D), jnp.float32),
            jax.ShapeDtypeStruct((NC, NP), jnp.float32),
        ],
        mesh=_sc_mesh(),
        scratch_types=[
            [pltpu.VMEM((G, CHUNK), jnp.int32) for _ in range(2)],
            [pltpu.VMEM((G, CHUNK), jnp.int32) for _ in range(2)],
            [pltpu.VMEM((CHUNK, D), jnp.float32) for _ in range(2)],
            [pltpu.SemaphoreType.DMA for _ in range(2)],
            [pltpu.SemaphoreType.DMA for _ in range(2)],
            [pltpu.VMEM((CHUNK,), jnp.float32) for _ in range(2)],
            [pltpu.SemaphoreType.DMA for _ in range(2)],
            pltpu.VMEM((ROWS_PER_TILE,), jnp.float32),
            pltpu.VMEM_SHARED((NP, D), jnp.float32),
            pltpu.VMEM_SHARED((NP,), jnp.float32),
        ],
    )(y, val, src2d, dst2d)


BLK = 1024
GRID = NP // BLK


def _row_spec():
    return pl.BlockSpec((BLK, D), lambda i: (i, 0))


def _part_spec():
    return pl.BlockSpec((NC, BLK, D), lambda i: (0, i, 0))


def _vec_spec():
    return pl.BlockSpec((BLK,), lambda i: (i,))


def _w_spec():
    return pl.BlockSpec((D, D), lambda i: (0, 0))


def _bias_spec():
    return pl.BlockSpec((1, D), lambda i: (0, 0))


def _pvec_spec():
    return pl.BlockSpec((NC, BLK), lambda i: (0, i))


def _prep_m_tc(lat, cond, w1, w2, w3a, w3b, m):
    wa = jnp.dot(w1[...], w3a[...], preferred_element_type=jnp.float32)
    wb = jnp.dot(w2[...], w3b[...], preferred_element_type=jnp.float32)
    m[...] = jnp.dot(lat[...], wa, preferred_element_type=jnp.float32) + jnp.dot(
        cond[...], wb, preferred_element_type=jnp.float32
    )


def _prep_m(lat, cond, w1, w2, w3a, w3b):
    return pl.pallas_call(
        _prep_m_tc,
        grid=(GRID,),
        in_specs=[
            _row_spec(), _row_spec(), _w_spec(), _w_spec(), _w_spec(),
            _w_spec(),
        ],
        out_specs=_row_spec(),
        out_shape=jax.ShapeDtypeStruct((NP, D), jnp.float32),
    )(lat, cond, w1, w2, w3a, w3b)


def _prep_d_tc(m, degp, y1, dinv, q):
    dp = degp[...]
    deg = dp[0] + dp[1] + 1.0
    di = lax.rsqrt(deg)
    dinv[...] = di
    q[...] = 1.0 / deg
    y1[...] = m[...] * di[:, None]


def _prep_d(m, degp):
    return pl.pallas_call(
        _prep_d_tc,
        grid=(GRID,),
        in_specs=[_row_spec(), _pvec_spec()],
        out_specs=[_row_spec(), _vec_spec(), _vec_spec()],
        out_shape=[
            jax.ShapeDtypeStruct((NP, D), jnp.float32),
            jax.ShapeDtypeStruct((NP,), jnp.float32),
            jax.ShapeDtypeStruct((NP,), jnp.float32),
        ],
    )(m, degp)


def _mid1_tc(p, sp, m, dinv, q, a1, y2, s1, ds1):
    di = dinv[...]
    qq = q[...]
    pv = p[...]
    spv = sp[...]
    a = di[:, None] * (pv[0] + pv[1]) + qq[:, None] * m[...]
    a1[...] = a
    y2[...] = a * di[:, None]
    sv = di * (spv[0] + spv[1]) + qq
    s1[...] = sv
    ds1[...] = di * sv


def _mid1(p, sp, m, dinv, q):
    return pl.pallas_call(
        _mid1_tc,
        grid=(GRID,),
        in_specs=[_part_spec(), _pvec_spec(), _row_spec(), _vec_spec(),
                  _vec_spec()],
        out_specs=[_row_spec(), _row_spec(), _vec_spec(), _vec_spec()],
        out_shape=[
            jax.ShapeDtypeStruct((NP, D), jnp.float32),
            jax.ShapeDtypeStruct((NP, D), jnp.float32),
            jax.ShapeDtypeStruct((NP,), jnp.float32),
            jax.ShapeDtypeStruct((NP,), jnp.float32),
        ],
    )(p, sp, m, dinv, q)


def _mid2_tc(p, sp, a1, s1, dinv, q, a2, y3, s2):
    di = dinv[...]
    qq = q[...]
    pv = p[...]
    spv = sp[...]
    a = di[:, None] * (pv[0] + pv[1]) + qq[:, None] * a1[...]
    a2[...] = a
    y3[...] = a * di[:, None]
    s2[...] = di * (spv[0] + spv[1]) + qq * s1[...]


def _mid2(p, sp, a1, s1, dinv, q):
    return pl.pallas_call(
        _mid2_tc,
        grid=(GRID,),
        in_specs=[_part_spec(), _pvec_spec(), _row_spec(), _vec_spec(),
                  _vec_spec(), _vec_spec()],
        out_specs=[_row_spec(), _row_spec(), _vec_spec()],
        out_shape=[
            jax.ShapeDtypeStruct((NP, D), jnp.float32),
            jax.ShapeDtypeStruct((NP, D), jnp.float32),
            jax.ShapeDtypeStruct((NP,), jnp.float32),
        ],
    )(p, sp, a1, s1, dinv, q)


def _final_tc(p, a2, s1, s2, dinv, q, w3a, w3b, w4, b1, b2, b3, b4, out):
    di = dinv[...]
    qq = q[...]
    pv = p[...]
    a3 = di[:, None] * (pv[0] + pv[1]) + qq[:, None] * a2[...]
    w4v = w4[...]
    bc = jnp.dot(b1[...], w3a[...], preferred_element_type=jnp.float32) + jnp.dot(
        b2[...], w3b[...], preferred_element_type=jnp.float32
    )
    bcw4 = jnp.dot(bc, w4v, preferred_element_type=jnp.float32)
    b3w4 = jnp.dot(b3[...], w4v, preferred_element_type=jnp.float32)
    out[...] = (
        jnp.dot(a3, w4v, preferred_element_type=jnp.float32)
        + s2[...][:, None] * bcw4
        + s1[...][:, None] * b3w4
        + b4[...]
    )


def _final(p, a2, s1, s2, dinv, q, w3a, w3b, w4, b1, b2, b3, b4):
    return pl.pallas_call(
        _final_tc,
        grid=(GRID,),
        in_specs=[
            _part_spec(), _row_spec(), _vec_spec(), _vec_spec(), _vec_spec(),
            _vec_spec(), _w_spec(), _w_spec(), _w_spec(), _bias_spec(),
            _bias_spec(), _bias_spec(), _bias_spec(),
        ],
        out_specs=_row_spec(),
        out_shape=jax.ShapeDtypeStruct((NP, D), jnp.float32),
    )(p, a2, s1, s2, dinv, q, w3a, w3b, w4, b1, b2, b3, b4)


def kernel(latent, condition, edge_index, W1, b1, W2, b2, W3, b3, W4, b4):
    src = edge_index[0]
    dst = edge_index[1]
    pad_idx = (N + jnp.arange(EP - E, dtype=jnp.int32) % (NP - N)).astype(jnp.int32)
    src2d = jnp.concatenate([src, pad_idx]).reshape(NCHUNKS, CHUNK)
    dst2d = jnp.concatenate([dst, pad_idx]).reshape(NCHUNKS, CHUNK)

    lat = jnp.zeros((NP, D), jnp.float32).at[:N].set(latent)
    cond = jnp.zeros((NP, D), jnp.float32).at[:N].set(condition)
    b1r = b1.reshape(1, D)
    b2r = b2.reshape(1, D)
    b3r = b3.reshape(1, D)
    b4r = b4.reshape(1, D)
    w3a = W3[:D]
    w3b = W3[D:]

    degp = _degree(dst2d)
    m = _prep_m(lat, cond, W1, W2, w3a, w3b)
    y1, dinv, q = _prep_d(m, degp)
    p1, sp1 = _prop_s(y1, dinv, src2d, dst2d)
    a1, y2, s1, ds1 = _mid1(p1, sp1, m, dinv, q)
    p2, sp2 = _prop_s(y2, ds1, src2d, dst2d)
    a2, y3, s2 = _mid2(p2, sp2, a1, s1, dinv, q)
    p3 = _prop(y3, src2d, dst2d)
    out = _final(p3, a2, s1, s2, dinv, q, w3a, w3b, W4, b1r, b2r, b3r, b4r)
    return out[:N]


# speedup vs baseline: 25.2709x; 1.0077x over previous
import jax
import jax.numpy as jnp
from jax import lax
from jax.experimental import pallas as pl
from jax.experimental.pallas import tpu as pltpu
from jax.experimental.pallas import tpu_sc as plsc

N = 10000
E = 320000
D = 128
NP = 10240
NC = 2
NS = 16
NW = NC * NS
CHUNK = 128
NCHUNKS = 2560
EP = NCHUNKS * CHUNK
CPT = NCHUNKS // NW
G = 8
NG = CPT // G
NBUF = 4
ROWS_PER_TILE = NP // NS
TILE_WB = ROWS_PER_TILE // CHUNK


def _sc_mesh():
    return plsc.VectorSubcoreMesh(
        core_axis_name="c", subcore_axis_name="s", num_cores=NC, num_subcores=NS
    )


def _deg_sc(dst_hbm, out_hbm, dst_v, ones_v, zrow_v, acc_sh):
    c = lax.axis_index("c")
    s = lax.axis_index("s")
    wid = s * NC + c
    pltpu.sync_copy(dst_hbm.at[pl.ds(wid * CPT, CPT)], dst_v)

    z = jnp.zeros((16,), jnp.float32)
    one = jnp.ones((16,), jnp.float32)
    for k in range(8):
        ones_v[pl.ds(k * 16, 16)] = one

    def zrow(i, _):
        zrow_v[pl.ds(i * 16, 16)] = z
        return 0

    lax.fori_loop(0, ROWS_PER_TILE // 16, zrow, 0)
    pltpu.sync_copy(zrow_v, acc_sh.at[pl.ds(s * ROWS_PER_TILE, ROWS_PER_TILE)])
    plsc.subcore_barrier()

    def chunk(j, _):
        pltpu.sync_copy(ones_v, acc_sh.at[dst_v.at[j]], add=True)
        return 0

    lax.fori_loop(0, CPT, chunk, 0)
    plsc.subcore_barrier()
    pltpu.sync_copy(
        acc_sh.at[pl.ds(s * ROWS_PER_TILE, ROWS_PER_TILE)],
        out_hbm.at[c, pl.ds(s * ROWS_PER_TILE, ROWS_PER_TILE)],
    )


def _degree(dst2d):
    return pl.kernel(
        _deg_sc,
        out_type=jax.ShapeDtypeStruct((NC, NP), jnp.float32),
        mesh=_sc_mesh(),
        scratch_types=[
            pltpu.VMEM((CPT, CHUNK), jnp.int32),
            pltpu.VMEM((CHUNK,), jnp.float32),
            pltpu.VMEM((ROWS_PER_TILE,), jnp.float32),
            pltpu.VMEM_SHARED((NP,), jnp.float32),
        ],
    )(dst2d)


def _make_prop_body(with_s):
    def body(y_hbm, *args):
        if with_s:
            (val_hbm, src_hbm, dst_hbm, out_hbm, sout_hbm, srcb, dstb, bufs,
             sems, isems, sbufs, ssems, sscat, zvec, acc_sh, sacc_sh) = args
        else:
            (src_hbm, dst_hbm, out_hbm, srcb, dstb, bufs, sems, isems,
             acc_sh) = args
        c = lax.axis_index("c")
        s = lax.axis_index("s")
        wid = s * NC + c
        base = wid * CPT

        pltpu.async_copy(src_hbm.at[pl.ds(base, G)], srcb[0], isems[0])
        pltpu.async_copy(dst_hbm.at[pl.ds(base, G)], dstb[0], isems[0])
        pltpu.async_copy(src_hbm.at[pl.ds(base + G, G)], srcb[1], isems[1])
        pltpu.async_copy(dst_hbm.at[pl.ds(base + G, G)], dstb[1], isems[1])

        z = jnp.zeros((16,), jnp.float32)

        def zrow(i, _):
            for k in range(8):
                bufs[0][i, pl.ds(k * 16, 16)] = z
            return 0

        lax.fori_loop(0, CHUNK, zrow, 0)
        for t in range(TILE_WB):
            pltpu.sync_copy(
                bufs[0], acc_sh.at[pl.ds(s * ROWS_PER_TILE + t * CHUNK, CHUNK)]
            )
        if with_s:
            def zv(i, _):
                zvec[pl.ds(i * 16, 16)] = z
                return 0

            lax.fori_loop(0, ROWS_PER_TILE // 16, zv, 0)
            pltpu.sync_copy(
                zvec, sacc_sh.at[pl.ds(s * ROWS_PER_TILE, ROWS_PER_TILE)]
            )

        pltpu.make_async_copy(src_hbm.at[pl.ds(base, G)], srcb[0],
                              isems[0]).wait()
        pltpu.make_async_copy(dst_hbm.at[pl.ds(base, G)], dstb[0],
                              isems[0]).wait()
        for b in range(2):
            pltpu.async_copy(y_hbm.at[srcb[0].at[b]], bufs[b], sems[b])
            if with_s:
                pltpu.async_copy(val_hbm.at[srcb[0].at[b]], sbufs[b], ssems[b])
        plsc.subcore_barrier()

        def pair(g2, _):
            for par in range(2):
                g = g2 * 2 + par
                for i in range(G):
                    b = i % 2
                    sb = i % 4
                    sbn = (i + 2) % 4
                    pltpu.make_async_copy(
                        y_hbm.at[srcb[par].at[i]], bufs[b], sems[b]
                    ).wait()
                    if with_s:
                        pltpu.make_async_copy(
                            val_hbm.at[srcb[par].at[i]], sbufs[sb], ssems[sb]
                        ).wait()
                    if i == G - 2:
                        def iwait():
                            pltpu.make_async_copy(
                                src_hbm.at[pl.ds(0, G)], srcb[1 - par],
                                isems[1 - par],
                            ).wait()
                            pltpu.make_async_copy(
                                dst_hbm.at[pl.ds(0, G)], dstb[1 - par],
                                isems[1 - par],
                            ).wait()
                        if par == 0:
                            iwait()
                        else:
                            pl.when(g2 < NG // 2 - 1)(iwait)
                    pltpu.sync_copy(bufs[b], acc_sh.at[dstb[par].at[i]],
                                    add=True)
                    if with_s:
                        pltpu.async_copy(sbufs[sb],
                                         sacc_sh.at[dstb[par].at[i]],
                                         sscat[sb], add=True)
                    if i < G - 2:
                        pltpu.async_copy(y_hbm.at[srcb[par].at[i + 2]],
                                         bufs[b], sems[b])
                        if with_s:
                            def sswait(sbn=sbn, par=par):
                                pltpu.make_async_copy(
                                    sbufs[sbn],
                                    sacc_sh.at[dstb[par].at[0]],
                                    sscat[sbn],
                                ).wait()
                            if par == 0 and i < 2:
                                pl.when(g2 > 0)(sswait)
                            else:
                                sswait()
                            pltpu.async_copy(val_hbm.at[srcb[par].at[i + 2]],
                                             sbufs[sbn], ssems[sbn])
                    else:
                        def nxt(b=b, i=i, par=par, sbn=sbn):
                            pltpu.async_copy(
                                y_hbm.at[srcb[1 - par].at[i + 2 - G]], bufs[b],
                                sems[b],
                            )
                            if with_s:
                                pltpu.make_async_copy(
                                    sbufs[sbn],
                                    sacc_sh.at[dstb[par].at[0]],
                                    sscat[sbn],
                                ).wait()
                                pltpu.async_copy(
                                    val_hbm.at[srcb[1 - par].at[i + 2 - G]],
                                    sbufs[sbn], ssems[sbn],
                                )
                        if par == 0:
                            nxt()
                        else:
                            pl.when(g2 < NG // 2 - 1)(nxt)
                def ild(par=par, g=g):
                    pltpu.async_copy(
                        src_hbm.at[pl.ds(base + (g + 2) * G, G)], srcb[par],
                        isems[par],
                    )
                    pltpu.async_copy(
                        dst_hbm.at[pl.ds(base + (g + 2) * G, G)], dstb[par],
                        isems[par],
                    )
                pl.when(g2 < NG // 2 - 1)(ild)
            return 0

        lax.fori_loop(0, NG // 2, pair, 0)
        if with_s:
            for k in range(4):
                pltpu.make_async_copy(
                    sbufs[k], sacc_sh.at[dstb[0].at[0]], sscat[k]
                ).wait()
        plsc.subcore_barrier()
        for t in range(TILE_WB):
            r0 = s * ROWS_PER_TILE + t * CHUNK
            pltpu.sync_copy(
                acc_sh.at[pl.ds(r0, CHUNK)], out_hbm.at[c, pl.ds(r0, CHUNK)]
            )
        if with_s:
            pltpu.sync_copy(
                sacc_sh.at[pl.ds(s * ROWS_PER_TILE, ROWS_PER_TILE)],
                sout_hbm.at[c, pl.ds(s * ROWS_PER_TILE, ROWS_PER_TILE)],
            )

    return body


def _prop(y, src2d, dst2d):
    return pl.kernel(
        _make_prop_body(False),
        out_type=jax.ShapeDtypeStruct((NC, NP, D), jnp.float32),
        mesh=_sc_mesh(),
        scratch_types=[
            [pltpu.VMEM((G, CHUNK), jnp.int32) for _ in range(2)],
            [pltpu.VMEM((G, CHUNK), jnp.int32) for _ in range(2)],
            [pltpu.VMEM((CHUNK, D), jnp.float32) for _ in range(2)],
            [pltpu.SemaphoreType.DMA for _ in range(2)],
            [pltpu.SemaphoreType.DMA for _ in range(2)],
            pltpu.VMEM_SHARED((NP, D), jnp.float32),
        ],
    )(y, src2d, dst2d)


def _prop_s(y, val, src2d, dst2d):
    return pl.kernel(
        _make_prop_body(True),
        out_type=[
            jax.ShapeDtypeStruct((NC, NP, D), jnp.float32),
            jax.ShapeDtypeStruct((NC, NP), jnp.float32),
        ],
        mesh=_sc_mesh(),
        scratch_types=[
            [pltpu.VMEM((G, CHUNK), jnp.int32) for _ in range(2)],
            [pltpu.VMEM((G, CHUNK), jnp.int32) for _ in range(2)],
            [pltpu.VMEM((CHUNK, D), jnp.float32) for _ in range(2)],
            [pltpu.SemaphoreType.DMA for _ in range(2)],
            [pltpu.SemaphoreType.DMA for _ in range(2)],
            [pltpu.VMEM((CHUNK,), jnp.float32) for _ in range(4)],
            [pltpu.SemaphoreType.DMA for _ in range(4)],
            [pltpu.SemaphoreType.DMA for _ in range(4)],
            pltpu.VMEM((ROWS_PER_TILE,), jnp.float32),
            pltpu.VMEM_SHARED((NP, D), jnp.float32),
            pltpu.VMEM_SHARED((NP,), jnp.float32),
        ],
    )(y, val, src2d, dst2d)


BLK = 1024
GRID = NP // BLK


def _row_spec():
    return pl.BlockSpec((BLK, D), lambda i: (i, 0))


def _part_spec():
    return pl.BlockSpec((NC, BLK, D), lambda i: (0, i, 0))


def _vec_spec():
    return pl.BlockSpec((BLK,), lambda i: (i,))


def _w_spec():
    return pl.BlockSpec((D, D), lambda i: (0, 0))


def _bias_spec():
    return pl.BlockSpec((1, D), lambda i: (0, 0))


def _pvec_spec():
    return pl.BlockSpec((NC, BLK), lambda i: (0, i))


def _prep_m_tc(lat, cond, w1, w2, w3a, w3b, m):
    wa = jnp.dot(w1[...], w3a[...], preferred_element_type=jnp.float32)
    wb = jnp.dot(w2[...], w3b[...], preferred_element_type=jnp.float32)
    m[...] = jnp.dot(lat[...], wa, preferred_element_type=jnp.float32) + jnp.dot(
        cond[...], wb, preferred_element_type=jnp.float32
    )


def _prep_m(lat, cond, w1, w2, w3a, w3b):
    return pl.pallas_call(
        _prep_m_tc,
        grid=(GRID,),
        in_specs=[
            _row_spec(), _row_spec(), _w_spec(), _w_spec(), _w_spec(),
            _w_spec(),
        ],
        out_specs=_row_spec(),
        out_shape=jax.ShapeDtypeStruct((NP, D), jnp.float32),
    )(lat, cond, w1, w2, w3a, w3b)


def _prep_d_tc(m, degp, y1, dinv, q):
    dp = degp[...]
    deg = dp[0] + dp[1] + 1.0
    di = lax.rsqrt(deg)
    dinv[...] = di
    q[...] = 1.0 / deg
    y1[...] = m[...] * di[:, None]


def _prep_d(m, degp):
    return pl.pallas_call(
        _prep_d_tc,
        grid=(GRID,),
        in_specs=[_row_spec(), _pvec_spec()],
        out_specs=[_row_spec(), _vec_spec(), _vec_spec()],
        out_shape=[
            jax.ShapeDtypeStruct((NP, D), jnp.float32),
            jax.ShapeDtypeStruct((NP,), jnp.float32),
            jax.ShapeDtypeStruct((NP,), jnp.float32),
        ],
    )(m, degp)


def _mid1_tc(p, sp, m, dinv, q, a1, y2, s1, ds1):
    di = dinv[...]
    qq = q[...]
    pv = p[...]
    spv = sp[...]
    a = di[:, None] * (pv[0] + pv[1]) + qq[:, None] * m[...]
    a1[...] = a
    y2[...] = a * di[:, None]
    sv = di * (spv[0] + spv[1]) + qq
    s1[...] = sv
    ds1[...] = di * sv


def _mid1(p, sp, m, dinv, q):
    return pl.pallas_call(
        _mid1_tc,
        grid=(GRID,),
        in_specs=[_part_spec(), _pvec_spec(), _row_spec(), _vec_spec(),
                  _vec_spec()],
        out_specs=[_row_spec(), _row_spec(), _vec_spec(), _vec_spec()],
        out_shape=[
            jax.ShapeDtypeStruct((NP, D), jnp.float32),
            jax.ShapeDtypeStruct((NP, D), jnp.float32),
            jax.ShapeDtypeStruct((NP,), jnp.float32),
            jax.ShapeDtypeStruct((NP,), jnp.float32),
        ],
    )(p, sp, m, dinv, q)


def _mid2_tc(p, sp, a1, s1, dinv, q, a2, y3, s2):
    di = dinv[...]
    qq = q[...]
    pv = p[...]
    spv = sp[...]
    a = di[:, None] * (pv[0] + pv[1]) + qq[:, None] * a1[...]
    a2[...] = a
    y3[...] = a * di[:, None]
    s2[...] = di * (spv[0] + spv[1]) + qq * s1[...]


def _mid2(p, sp, a1, s1, dinv, q):
    return pl.pallas_call(
        _mid2_tc,
        grid=(GRID,),
        in_specs=[_part_spec(), _pvec_spec(), _row_spec(), _vec_spec(),
                  _vec_spec(), _vec_spec()],
        out_specs=[_row_spec(), _row_spec(), _vec_spec()],
        out_shape=[
            jax.ShapeDtypeStruct((NP, D), jnp.float32),
            jax.ShapeDtypeStruct((NP, D), jnp.float32),
            jax.ShapeDtypeStruct((NP,), jnp.float32),
        ],
    )(p, sp, a1, s1, dinv, q)


def _final_tc(p, a2, s1, s2, dinv, q, w3a, w3b, w4, b1, b2, b3, b4, out):
    di = dinv[...]
    qq = q[...]
    pv = p[...]
    a3 = di[:, None] * (pv[0] + pv[1]) + qq[:, None] * a2[...]
    w4v = w4[...]
    bc = jnp.dot(b1[...], w3a[...], preferred_element_type=jnp.float32) + jnp.dot(
        b2[...], w3b[...], preferred_element_type=jnp.float32
    )
    bcw4 = jnp.dot(bc, w4v, preferred_element_type=jnp.float32)
    b3w4 = jnp.dot(b3[...], w4v, preferred_element_type=jnp.float32)
    out[...] = (
        jnp.dot(a3, w4v, preferred_element_type=jnp.float32)
        + s2[...][:, None] * bcw4
        + s1[...][:, None] * b3w4
        + b4[...]
    )


def _final(p, a2, s1, s2, dinv, q, w3a, w3b, w4, b1, b2, b3, b4):
    return pl.pallas_call(
        _final_tc,
        grid=(GRID,),
        in_specs=[
            _part_spec(), _row_spec(), _vec_spec(), _vec_spec(), _vec_spec(),
            _vec_spec(), _w_spec(), _w_spec(), _w_spec(), _bias_spec(),
            _bias_spec(), _bias_spec(), _bias_spec(),
        ],
        out_specs=_row_spec(),
        out_shape=jax.ShapeDtypeStruct((NP, D), jnp.float32),
    )(p, a2, s1, s2, dinv, q, w3a, w3b, w4, b1, b2, b3, b4)


def kernel(latent, condition, edge_index, W1, b1, W2, b2, W3, b3, W4, b4):
    src = edge_index[0]
    dst = edge_index[1]
    pad_idx = (N + jnp.arange(EP - E, dtype=jnp.int32) % (NP - N)).astype(jnp.int32)
    src2d = jnp.concatenate([src, pad_idx]).reshape(NCHUNKS, CHUNK)
    dst2d = jnp.concatenate([dst, pad_idx]).reshape(NCHUNKS, CHUNK)

    lat = jnp.zeros((NP, D), jnp.float32).at[:N].set(latent)
    cond = jnp.zeros((NP, D), jnp.float32).at[:N].set(condition)
    b1r = b1.reshape(1, D)
    b2r = b2.reshape(1, D)
    b3r = b3.reshape(1, D)
    b4r = b4.reshape(1, D)
    w3a = W3[:D]
    w3b = W3[D:]

    degp = _degree(dst2d)
    m = _prep_m(lat, cond, W1, W2, w3a, w3b)
    y1, dinv, q = _prep_d(m, degp)
    p1, sp1 = _prop_s(y1, dinv, src2d, dst2d)
    a1, y2, s1, ds1 = _mid1(p1, sp1, m, dinv, q)
    p2, sp2 = _prop_s(y2, ds1, src2d, dst2d)
    a2, y3, s2 = _mid2(p2, sp2, a1, s1, dinv, q)
    p3 = _prop(y3, src2d, dst2d)
    out = _final(p3, a2, s1, s2, dinv, q, w3a, w3b, W4, b1r, b2r, b3r, b4r)
    return out[:N]
